# baseline (device time: 51770 ns/iter reference)
import jax
import jax.numpy as jnp
from jax import lax
from jax.experimental import pallas as pl
from jax.experimental.pallas import tpu as pltpu

N_DEV = 16
SQ = 512
D_MODEL = 1024
SKV = 2048
H_LOCAL = 8
GQA = 4
KV_LOCAL = H_LOCAL // GQA
DH = 128
SCALE = 0.08838834764831843

CHUNK = SQ // N_DEV


def kernel(x, Wq, Wo, K_ext, V_ext):
    def body(x_ref, wq_ref, wo_ref, kext_ref, vext_ref, out_ref,
             kbuf, vbuf, kv_sems, sendb, agb, rs_recv, ag_recv,
             rs_send_sems, rs_recv_sems, ag_send_sems, ag_recv_sems):
        m = lax.axis_index("i")

        barrier_sem = pltpu.get_barrier_semaphore()
        for t in range(1, N_DEV):
            d = lax.rem(m + t, N_DEV)
            pl.semaphore_signal(
                barrier_sem, inc=1,
                device_id=(d,), device_id_type=pl.DeviceIdType.MESH,
            )

        copies = []
        for j in range(KV_LOCAL):
            h = m * KV_LOCAL + j
            ck = pltpu.make_async_copy(
                kext_ref.at[0, :, h, :], kbuf.at[j], kv_sems.at[2 * j])
            cv = pltpu.make_async_copy(
                vext_ref.at[0, :, h, :], vbuf.at[j], kv_sems.at[2 * j + 1])
            ck.start()
            cv.start()
            copies += [ck, cv]

        q = jnp.dot(x_ref[:], wq_ref[:], preferred_element_type=jnp.float32)

        for c in copies:
            c.wait()

        outs = []
        for h in range(H_LOCAL):
            qh = q[:, h * DH:(h + 1) * DH]
            kv = h // GQA
            s = lax.dot_general(
                qh, kbuf[kv],
                (((1,), (1,)), ((), ())),
                preferred_element_type=jnp.float32,
            ) * SCALE
            mx = jnp.max(s, axis=1, keepdims=True)
            p = jnp.exp(s - mx)
            l = jnp.sum(p, axis=1, keepdims=True)
            oh = jnp.dot(p, vbuf[kv], preferred_element_type=jnp.float32) / l
            outs.append(oh)
        attn = jnp.concatenate(outs, axis=1)
        out_ref[:] = jnp.dot(attn, wo_ref[:],
                             preferred_element_type=jnp.float32)

        import os as _os
        if _os.environ.get("DISABLE_COMM"):
            return

        sendb[:] = out_ref[:].astype(jnp.bfloat16)
        pl.semaphore_wait(barrier_sem, N_DEV - 1)
        rs_descs = []
        for t in range(1, N_DEV):
            d = lax.rem(m + t, N_DEV)
            rdma = pltpu.make_async_remote_copy(
                src_ref=sendb.at[pl.ds(pl.multiple_of(d * CHUNK, 32), CHUNK), :],
                dst_ref=rs_recv.at[m],
                send_sem=rs_send_sems.at[d],
                recv_sem=rs_recv_sems.at[m],
                device_id=(d,),
                device_id_type=pl.DeviceIdType.MESH,
            )
            rdma.start()
            rs_descs.append(rdma)

        my_lo = pl.multiple_of(m * CHUNK, 32)
        acc = out_ref[pl.ds(my_lo, CHUNK), :]
        for t in range(1, N_DEV):
            j = lax.rem(m - t + N_DEV, N_DEV)
            recv = pltpu.make_async_remote_copy(
                src_ref=sendb.at[pl.ds(0, CHUNK), :],
                dst_ref=rs_recv.at[j],
                send_sem=rs_send_sems.at[0],
                recv_sem=rs_recv_sems.at[j],
                device_id=(j,),
                device_id_type=pl.DeviceIdType.MESH,
            )
            recv.wait_recv()
            acc = acc + rs_recv[j].astype(jnp.float32)
        out_ref[pl.ds(my_lo, CHUNK), :] = acc

        agb[:] = acc.astype(jnp.bfloat16)
        ag_descs = []
        for t in range(1, N_DEV):
            d = lax.rem(m + t, N_DEV)
            rdma = pltpu.make_async_remote_copy(
                src_ref=agb,
                dst_ref=ag_recv.at[m],
                send_sem=ag_send_sems.at[d],
                recv_sem=ag_recv_sems.at[m],
                device_id=(d,),
                device_id_type=pl.DeviceIdType.MESH,
            )
            rdma.start()
            ag_descs.append(rdma)

        for t in range(1, N_DEV):
            j = lax.rem(m - t + N_DEV, N_DEV)
            recv = pltpu.make_async_remote_copy(
                src_ref=agb,
                dst_ref=ag_recv.at[j],
                send_sem=ag_send_sems.at[0],
                recv_sem=ag_recv_sems.at[j],
                device_id=(j,),
                device_id_type=pl.DeviceIdType.MESH,
            )
            recv.wait_recv()
            out_ref[pl.ds(pl.multiple_of(j * CHUNK, 32), CHUNK), :] = (
                ag_recv[j].astype(jnp.float32)
            )

        for rdma in rs_descs + ag_descs:
            rdma.wait_send()

    out = pl.pallas_call(
        body,
        out_shape=jax.ShapeDtypeStruct((SQ, D_MODEL), jnp.float32),
        in_specs=[
            pl.BlockSpec(memory_space=pltpu.VMEM),
            pl.BlockSpec(memory_space=pltpu.VMEM),
            pl.BlockSpec(memory_space=pltpu.VMEM),
            pl.BlockSpec(memory_space=pl.ANY),
            pl.BlockSpec(memory_space=pl.ANY),
        ],
        out_specs=pl.BlockSpec(memory_space=pltpu.VMEM),
        scratch_shapes=[
            pltpu.VMEM((KV_LOCAL, SKV, DH), jnp.float32),
            pltpu.VMEM((KV_LOCAL, SKV, DH), jnp.float32),
            pltpu.SemaphoreType.DMA((2 * KV_LOCAL,)),
            pltpu.VMEM((SQ, D_MODEL), jnp.bfloat16),
            pltpu.VMEM((CHUNK, D_MODEL), jnp.bfloat16),
            pltpu.VMEM((N_DEV, CHUNK, D_MODEL), jnp.bfloat16),
            pltpu.VMEM((N_DEV, CHUNK, D_MODEL), jnp.bfloat16),
            pltpu.SemaphoreType.DMA((N_DEV,)),
            pltpu.SemaphoreType.DMA((N_DEV,)),
            pltpu.SemaphoreType.DMA((N_DEV,)),
            pltpu.SemaphoreType.DMA((N_DEV,)),
        ],
        compiler_params=pltpu.CompilerParams(
            vmem_limit_bytes=96 * 1024 * 1024,
            collective_id=0,
        ),
    )(x[0], Wq, Wo, K_ext, V_ext)
    return out[None]
